# baseline (speedup 1.0000x reference)
import functools

import jax
import jax.numpy as jnp
from jax import lax
from jax.experimental import pallas as pl
from jax.experimental.pallas import tpu as pltpu
from jax.experimental.pallas import tpu_sc as plsc

BATCH = 16384
DIM = 128
HALF = DIM // 2
LANES = 16
NC = 2
NS = 16
NW = NC * NS
ROWS_PER_W = BATCH // NW
CHUNK = 64
NCHUNK = ROWS_PER_W // CHUNK
NBUF = 2


def _body(emb_hbm, idx_hbm, real_hbm, imag_hbm, out_hbm,
          idx_v0, idx_v1, gidx_v0, gidx_v1, emb_v0, emb_v1,
          gr_v0, gr_v1, gi_v0, gi_v1, out_v0, out_v1,
          in_sem0, in_sem1, out_sem0, out_sem1):
    wid = lax.axis_index("s") * NC + lax.axis_index("c")
    idx_v = (idx_v0, idx_v1)
    gidx_v = (gidx_v0, gidx_v1)
    emb_v = (emb_v0, emb_v1)
    gr_v = (gr_v0, gr_v1)
    gi_v = (gi_v0, gi_v1)
    out_v = (out_v0, out_v1)
    in_sem = (in_sem0, in_sem1)
    out_sem = (out_sem0, out_sem1)

    def issue(c):
        b = c % NBUF
        base = wid * ROWS_PER_W + c * CHUNK
        pltpu.sync_copy(idx_hbm.at[pl.ds(base, CHUNK)],
                        idx_v[b].at[pl.ds(0, CHUNK)])
        for k in range(CHUNK // LANES):
            v = idx_v[b][pl.ds(k * LANES, LANES)]
            gidx_v[b][pl.ds(k * LANES, LANES)] = v >> 1
        return (
            pltpu.async_copy(emb_hbm.at[pl.ds(base, CHUNK)], emb_v[b],
                             in_sem[b]),
            pltpu.async_copy(real_hbm.at[gidx_v[b]], gr_v[b], in_sem[b]),
            pltpu.async_copy(imag_hbm.at[gidx_v[b]], gi_v[b], in_sem[b]),
        )

    pending = {0: issue(0)}
    out_pending = {}
    for c in range(NCHUNK):
        b = c % NBUF
        if c + 1 < NCHUNK:
            pending[c + 1] = issue(c + 1)
        for h in pending.pop(c):
            h.wait()
        if c - NBUF in out_pending:
            out_pending.pop(c - NBUF).wait()
        ev, rv, iv, ov, xv = emb_v[b], gr_v[b], gi_v[b], out_v[b], idx_v[b]

        def row(i, carry):
            off = (xv[pl.ds(i, LANES)][0] & 1) * HALF
            for j in range(HALF // LANES):
                er = ev[i, pl.ds(j * LANES, LANES)]
                ei = ev[i, pl.ds(HALF + j * LANES, LANES)]
                rr = rv[i, pl.ds(off + j * LANES, LANES)]
                ri = iv[i, pl.ds(off + j * LANES, LANES)]
                ov[i, pl.ds(j * LANES, LANES)] = er * rr - ei * ri
                ov[i, pl.ds(HALF + j * LANES, LANES)] = er * ri + ei * rr
            return carry

        lax.fori_loop(0, CHUNK, row, 0)
        base = wid * ROWS_PER_W + c * CHUNK
        out_pending[c] = pltpu.async_copy(
            ov, out_hbm.at[pl.ds(base, CHUNK)], out_sem[b])
    for h in out_pending.values():
        h.wait()


_sc_call = functools.partial(
    pl.kernel,
    out_type=jax.ShapeDtypeStruct((BATCH, DIM), jnp.float32),
    mesh=plsc.VectorSubcoreMesh(core_axis_name="c", subcore_axis_name="s"),
    scratch_types=[
        pltpu.VMEM((CHUNK + LANES,), jnp.int32),
        pltpu.VMEM((CHUNK + LANES,), jnp.int32),
        pltpu.VMEM((CHUNK,), jnp.int32),
        pltpu.VMEM((CHUNK,), jnp.int32),
        pltpu.VMEM((CHUNK, DIM), jnp.float32),
        pltpu.VMEM((CHUNK, DIM), jnp.float32),
        pltpu.VMEM((CHUNK, DIM), jnp.float32),
        pltpu.VMEM((CHUNK, DIM), jnp.float32),
        pltpu.VMEM((CHUNK, DIM), jnp.float32),
        pltpu.VMEM((CHUNK, DIM), jnp.float32),
        pltpu.VMEM((CHUNK, DIM), jnp.float32),
        pltpu.VMEM((CHUNK, DIM), jnp.float32),
        pltpu.SemaphoreType.DMA,
        pltpu.SemaphoreType.DMA,
        pltpu.SemaphoreType.DMA,
        pltpu.SemaphoreType.DMA,
    ],
)(_body)


def kernel(emb, rel_index, real, imag):
    idx = rel_index.astype(jnp.int32)
    real2 = real.reshape(real.shape[0] // 2, DIM)
    imag2 = imag.reshape(imag.shape[0] // 2, DIM)
    return _sc_call(emb, idx, real2, imag2)


# speedup vs baseline: 1.5866x; 1.5866x over previous
import functools

import jax
import jax.numpy as jnp
from jax import lax
from jax.experimental import pallas as pl
from jax.experimental.pallas import tpu as pltpu
from jax.experimental.pallas import tpu_sc as plsc

BATCH = 16384
DIM = 128
HALF = DIM // 2
LANES = 16
NC = 2
NS = 16
NW = NC * NS
ROWS_PER_W = BATCH // NW
CHUNK = 64
NCHUNK = ROWS_PER_W // CHUNK
NBUF = 2


def _body(emb_hbm, idx_hbm, real_hbm, imag_hbm, out_hbm,
          idx_v0, idx_v1, emb_v0, emb_v1,
          gr_v0, gr_v1, gi_v0, gi_v1, out_v0, out_v1,
          emb_sem0, emb_sem1, g_sem0, g_sem1, out_sem0, out_sem1):
    wid = lax.axis_index("s") * NC + lax.axis_index("c")
    idx_v = (idx_v0, idx_v1)
    emb_v = (emb_v0, emb_v1)
    gr_v = (gr_v0, gr_v1)
    gi_v = (gi_v0, gi_v1)
    out_v = (out_v0, out_v1)
    emb_sem = (emb_sem0, emb_sem1)
    g_sem = (g_sem0, g_sem1)
    out_sem = (out_sem0, out_sem1)

    def issue(c):
        b = c % NBUF
        base = wid * ROWS_PER_W + c * CHUNK
        pltpu.sync_copy(idx_hbm.at[pl.ds(base, CHUNK)],
                        idx_v[b].at[pl.ds(0, CHUNK)])
        h = pltpu.async_copy(emb_hbm.at[pl.ds(base, CHUNK)], emb_v[b],
                             emb_sem[b])

        def fire(i, carry):
            r = idx_v[b][pl.ds(i, LANES)][0]
            pltpu.make_async_copy(real_hbm.at[pl.ds(r, 1)],
                                  gr_v[b].at[pl.ds(i, 1)], g_sem[b]).start()
            pltpu.make_async_copy(imag_hbm.at[pl.ds(r, 1)],
                                  gi_v[b].at[pl.ds(i, 1)], g_sem[b]).start()
            return carry

        lax.fori_loop(0, CHUNK, fire, 0)
        return h

    def drain(c):
        b = c % NBUF

        def wait_row(i, carry):
            pltpu.make_async_copy(real_hbm.at[pl.ds(0, 1)],
                                  gr_v[b].at[pl.ds(i, 1)], g_sem[b]).wait()
            pltpu.make_async_copy(imag_hbm.at[pl.ds(0, 1)],
                                  gi_v[b].at[pl.ds(i, 1)], g_sem[b]).wait()
            return carry

        lax.fori_loop(0, CHUNK, wait_row, 0)

    pending = {0: issue(0)}
    out_pending = {}
    for c in range(NCHUNK):
        b = c % NBUF
        if c + 1 < NCHUNK:
            pending[c + 1] = issue(c + 1)
        pending.pop(c).wait()
        drain(c)
        if c - NBUF in out_pending:
            out_pending.pop(c - NBUF).wait()
        ev, rv, iv, ov = emb_v[b], gr_v[b], gi_v[b], out_v[b]

        def row(i, carry):
            for j in range(HALF // LANES):
                er = ev[i, pl.ds(j * LANES, LANES)]
                ei = ev[i, pl.ds(HALF + j * LANES, LANES)]
                rr = rv[i, pl.ds(j * LANES, LANES)]
                ri = iv[i, pl.ds(j * LANES, LANES)]
                ov[i, pl.ds(j * LANES, LANES)] = er * rr - ei * ri
                ov[i, pl.ds(HALF + j * LANES, LANES)] = er * ri + ei * rr
            return carry

        lax.fori_loop(0, CHUNK, row, 0)
        base = wid * ROWS_PER_W + c * CHUNK
        out_pending[c] = pltpu.async_copy(
            ov, out_hbm.at[pl.ds(base, CHUNK)], out_sem[b])
    for h in out_pending.values():
        h.wait()


_sc_call = functools.partial(
    pl.kernel,
    out_type=jax.ShapeDtypeStruct((BATCH, DIM), jnp.float32),
    mesh=plsc.VectorSubcoreMesh(core_axis_name="c", subcore_axis_name="s"),
    scratch_types=[
        pltpu.VMEM((CHUNK + LANES,), jnp.int32),
        pltpu.VMEM((CHUNK + LANES,), jnp.int32),
        pltpu.VMEM((CHUNK, DIM), jnp.float32),
        pltpu.VMEM((CHUNK, DIM), jnp.float32),
        pltpu.VMEM((CHUNK, HALF), jnp.float32),
        pltpu.VMEM((CHUNK, HALF), jnp.float32),
        pltpu.VMEM((CHUNK, HALF), jnp.float32),
        pltpu.VMEM((CHUNK, HALF), jnp.float32),
        pltpu.VMEM((CHUNK, DIM), jnp.float32),
        pltpu.VMEM((CHUNK, DIM), jnp.float32),
        pltpu.SemaphoreType.DMA,
        pltpu.SemaphoreType.DMA,
        pltpu.SemaphoreType.DMA,
        pltpu.SemaphoreType.DMA,
        pltpu.SemaphoreType.DMA,
        pltpu.SemaphoreType.DMA,
    ],
)(_body)


def kernel(emb, rel_index, real, imag):
    idx = rel_index.astype(jnp.int32)
    return _sc_call(emb, idx, real, imag)


# speedup vs baseline: 1.5995x; 1.0081x over previous
import functools

import jax
import jax.numpy as jnp
from jax import lax
from jax.experimental import pallas as pl
from jax.experimental.pallas import tpu as pltpu
from jax.experimental.pallas import tpu_sc as plsc

BATCH = 16384
DIM = 128
HALF = DIM // 2
LANES = 16
NC = 2
NS = 16
NW = NC * NS
ROWS_PER_W = BATCH // NW
CHUNK = 64
NCHUNK = ROWS_PER_W // CHUNK
NBUF = 2


def _body(emb_hbm, idx_hbm, real_hbm, imag_hbm, out_hbm,
          idx_v, emb_v0, emb_v1,
          gr_v0, gr_v1, gi_v0, gi_v1, out_v0, out_v1,
          emb_sem0, emb_sem1, gr_sem0, gr_sem1, gi_sem0, gi_sem1,
          out_sem0, out_sem1):
    wid = lax.axis_index("s") * NC + lax.axis_index("c")
    emb_v = (emb_v0, emb_v1)
    gr_v = (gr_v0, gr_v1)
    gi_v = (gi_v0, gi_v1)
    out_v = (out_v0, out_v1)
    emb_sem = (emb_sem0, emb_sem1)
    gr_sem = (gr_sem0, gr_sem1)
    gi_sem = (gi_sem0, gi_sem1)
    out_sem = (out_sem0, out_sem1)

    pltpu.sync_copy(idx_hbm.at[pl.ds(wid * ROWS_PER_W, ROWS_PER_W)],
                    idx_v.at[pl.ds(0, ROWS_PER_W)])

    def issue(c):
        b = c % NBUF
        base = wid * ROWS_PER_W + c * CHUNK
        h = pltpu.async_copy(emb_hbm.at[pl.ds(base, CHUNK)], emb_v[b],
                             emb_sem[b])

        def fire(i, carry):
            r = idx_v[pl.ds(c * CHUNK + i, LANES)][0]
            pltpu.make_async_copy(real_hbm.at[pl.ds(r, 1)],
                                  gr_v[b].at[pl.ds(i, 1)], gr_sem[b]).start()
            pltpu.make_async_copy(imag_hbm.at[pl.ds(r, 1)],
                                  gi_v[b].at[pl.ds(i, 1)], gi_sem[b]).start()
            return carry

        lax.fori_loop(0, CHUNK, fire, 0)
        return h

    def drain(c):
        b = c % NBUF

        def wait_row(i, carry):
            pltpu.make_async_copy(real_hbm.at[pl.ds(0, 1)],
                                  gr_v[b].at[pl.ds(i, 1)], gr_sem[b]).wait()
            pltpu.make_async_copy(imag_hbm.at[pl.ds(0, 1)],
                                  gi_v[b].at[pl.ds(i, 1)], gi_sem[b]).wait()
            return carry

        lax.fori_loop(0, CHUNK, wait_row, 0)

    pending = {0: issue(0)}
    out_pending = {}
    for c in range(NCHUNK):
        b = c % NBUF
        if c + 1 < NCHUNK:
            pending[c + 1] = issue(c + 1)
        pending.pop(c).wait()
        drain(c)
        if c - NBUF in out_pending:
            out_pending.pop(c - NBUF).wait()
        ev, rv, iv, ov = emb_v[b], gr_v[b], gi_v[b], out_v[b]

        def row(i, carry):
            for j in range(HALF // LANES):
                er = ev[i, pl.ds(j * LANES, LANES)]
                ei = ev[i, pl.ds(HALF + j * LANES, LANES)]
                rr = rv[i, pl.ds(j * LANES, LANES)]
                ri = iv[i, pl.ds(j * LANES, LANES)]
                ov[i, pl.ds(j * LANES, LANES)] = er * rr - ei * ri
                ov[i, pl.ds(HALF + j * LANES, LANES)] = er * ri + ei * rr
            return carry

        lax.fori_loop(0, CHUNK, row, 0)
        base = wid * ROWS_PER_W + c * CHUNK
        out_pending[c] = pltpu.async_copy(
            ov, out_hbm.at[pl.ds(base, CHUNK)], out_sem[b])
    for h in out_pending.values():
        h.wait()


_sc_call = functools.partial(
    pl.kernel,
    out_type=jax.ShapeDtypeStruct((BATCH, DIM), jnp.float32),
    mesh=plsc.VectorSubcoreMesh(core_axis_name="c", subcore_axis_name="s"),
    scratch_types=[
        pltpu.VMEM((ROWS_PER_W + LANES,), jnp.int32),
        pltpu.VMEM((CHUNK, DIM), jnp.float32),
        pltpu.VMEM((CHUNK, DIM), jnp.float32),
        pltpu.VMEM((CHUNK, HALF), jnp.float32),
        pltpu.VMEM((CHUNK, HALF), jnp.float32),
        pltpu.VMEM((CHUNK, HALF), jnp.float32),
        pltpu.VMEM((CHUNK, HALF), jnp.float32),
        pltpu.VMEM((CHUNK, DIM), jnp.float32),
        pltpu.VMEM((CHUNK, DIM), jnp.float32),
        pltpu.SemaphoreType.DMA,
        pltpu.SemaphoreType.DMA,
        pltpu.SemaphoreType.DMA,
        pltpu.SemaphoreType.DMA,
        pltpu.SemaphoreType.DMA,
        pltpu.SemaphoreType.DMA,
        pltpu.SemaphoreType.DMA,
        pltpu.SemaphoreType.DMA,
    ],
)(_body)


def kernel(emb, rel_index, real, imag):
    idx = rel_index.astype(jnp.int32)
    return _sc_call(emb, idx, real, imag)
